# baseline (speedup 1.0000x reference)
import functools

import numpy as np

import jax
import jax.numpy as jnp
from jax import lax
from jax.experimental import pallas as pl
from jax.experimental.pallas import tpu as pltpu
from jax.experimental.pallas import tpu_sc as plsc

NN = 10
S = NN + 1
NC, NS = 2, 16
NW = NC * NS
LANES = 16

PASS_ROWS = 5120
NPASS = 2


def _build_gather_sum(rows, d, ch, co, cip):
    dw = d // 2
    mesh = plsc.VectorSubcoreMesh(
        core_axis_name="c", subcore_axis_name="s", num_cores=NC,
        num_subcores=NS)
    rw = rows // NW
    nseg = 4
    cseg = ch // nseg
    sr = rw // nseg
    stg = PASS_ROWS // NS

    @functools.partial(
        pl.kernel,
        out_type=jax.ShapeDtypeStruct((rows, d), jnp.float32),
        mesh=mesh,
        scratch_types=[
            pltpu.VMEM((ch, cip), jnp.int32),
            pltpu.VMEM((2, cip), jnp.int32),
            pltpu.VMEM((2, cip, dw), jnp.int32),
            pltpu.VMEM((sr, d), jnp.float32),
            pltpu.VMEM_SHARED((PASS_ROWS, dw), jnp.int32),
            pltpu.SemaphoreType.DMA,
        ],
        compiler_params=pltpu.CompilerParams(needs_layout_passes=False),
    )
    def gather_sum(table, g3, out, idx_v, lidx_v, rows_v, acc_v, spm, sem):
        sid = lax.axis_index("s")
        w = sid * NC + lax.axis_index("c")
        pltpu.sync_copy(g3.at[w], idx_v)

        def remap_fire(c, h):
            bsl = c & 1
            for k in range(cip // LANES):
                seg = pl.ds(k * LANES, LANES)
                v = idx_v[c, seg] - h * PASS_ROWS
                ok = (v >= 0) & (v < PASS_ROWS)
                lidx_v[bsl, seg] = jnp.where(ok, v, 0)
            pltpu.async_copy(
                spm.at[lidx_v.at[bsl]], rows_v.at[bsl], sem)

        def segment(sg, carry0):
            def zero(i, carry):
                for g in range(d // LANES):
                    acc_v[i, pl.ds(g * LANES, LANES)] = jnp.zeros(
                        (LANES,), jnp.float32)
                return carry

            lax.fori_loop(0, sr, zero, 0, unroll=False)

            def passes(h, carry):
                pltpu.sync_copy(
                    table.at[pl.ds(h * PASS_ROWS + sid * stg, stg)],
                    spm.at[pl.ds(sid * stg, stg)])
                plsc.subcore_barrier()
                remap_fire(sg * cseg, h)

                def chunk(cl, carry2):
                    c = sg * cseg + cl
                    bsl = c & 1
                    pltpu.make_async_copy(
                        table.at[pl.ds(0, cip)], rows_v.at[bsl],
                        sem).wait()

                    @pl.when(cl + 1 < cseg)
                    def _():
                        remap_fire(c + 1, h)

                    def seg_loop(g, carry3):
                        src = pl.ds(g * LANES, LANES)
                        dst_a = pl.ds(g * 2 * LANES, LANES)
                        dst_b = pl.ds(g * 2 * LANES + LANES, LANES)
                        for o in range(co):
                            b0 = o * S
                            a, bb = plsc.unpack(
                                plsc.bitcast(
                                    rows_v[bsl, b0, src], jnp.bfloat16),
                                format=plsc.PackFormat.INTERLEAVED,
                                preferred_element_type=jnp.float32)
                            for r in range(1, S):
                                ar, br = plsc.unpack(
                                    plsc.bitcast(
                                        rows_v[bsl, b0 + r, src],
                                        jnp.bfloat16),
                                    format=plsc.PackFormat.INTERLEAVED,
                                    preferred_element_type=jnp.float32)
                                a = a + ar
                                bb = bb + br
                            oi = cl * co + o
                            acc_v[oi, dst_a] = acc_v[oi, dst_a] + a
                            acc_v[oi, dst_b] = acc_v[oi, dst_b] + bb
                        return carry3

                    lax.fori_loop(0, dw // LANES, seg_loop, 0,
                                  unroll=False)
                    return carry2

                lax.fori_loop(0, cseg, chunk, 0, unroll=False)
                plsc.subcore_barrier()
                return carry

            lax.fori_loop(0, NPASS, passes, 0, unroll=False)
            pltpu.sync_copy(
                acc_v, out.at[pl.ds(w * rw + sg * sr, sr)])
            return carry0

        lax.fori_loop(0, nseg, segment, 0, unroll=False)

    return gather_sum


def _unpack_perm(d):
    perm = []
    for base in range(0, d, 32):
        perm.extend(base + 2 * j for j in range(16))
        perm.extend(base + 2 * j + 1 for j in range(16))
    return np.asarray(perm, dtype=np.int32)


def _dense_body(s, blk, sums_ref, sc0_ref, sc1_ref, w1_ref, w2_ref, out_ref):
    w1 = w1_ref[...]
    acc = jnp.zeros((blk, out_ref.shape[1]), jnp.float32)
    for j in range(s):
        m = sums_ref[j] * sc0_ref[j]
        h = jnp.maximum(jnp.dot(m, w1, preferred_element_type=jnp.float32), 0.0)
        nrm = jnp.sqrt(jnp.sum(h * h, axis=1, keepdims=True))
        acc = acc + h / jnp.maximum(nrm, 1e-12)
    mean1 = acc * sc1_ref[...]
    h2 = jnp.maximum(
        jnp.dot(mean1, w2_ref[...], preferred_element_type=jnp.float32), 0.0)
    n2 = jnp.sqrt(jnp.sum(h2 * h2, axis=1, keepdims=True))
    out_ref[...] = h2 / jnp.maximum(n2, 1e-12)


def kernel(feats, adj0, adj1, samples, W1, W2):
    n, d = feats.shape
    e = adj0.shape[0]
    b = samples.shape[0]
    rows = b * S
    z = n

    starts = jnp.concatenate(
        [jnp.zeros((1,), adj1.dtype), jnp.cumsum(adj1)[:-1]])
    zt = jnp.stack([adj1, starts], axis=1)
    adj08 = adj0.reshape(-1, 8)
    tri = jnp.arange(3, dtype=starts.dtype)
    ar = jnp.arange(NN, dtype=adj1.dtype)

    def windows(st):
        trip = adj08[st[:, None] // 8 + tri[None, :]].reshape(-1, 24)
        r = st % 8
        w = trip[:, 0:NN]
        for j in range(1, 8):
            w = jnp.where((r == j)[:, None], trip[:, j:j + NN], w)
        return w

    g1 = zt[samples]
    size1, st1 = g1[:, 0], g1[:, 1]
    k1 = jnp.minimum(size1, NN)
    neigh1 = windows(st1)
    valid1 = ar[None, :] < k1[:, None]
    slots = jnp.where(valid1, neigh1, z)
    node_t = jnp.concatenate([slots.T, samples[None, :]], axis=0)
    flat = node_t.reshape(-1)
    is_z = flat == z
    fc = jnp.clip(flat, 0, n - 1)
    g2 = zt[fc]
    size2 = jnp.where(is_z, 0, g2[:, 0])
    k2 = jnp.minimum(size2, NN)
    st2 = jnp.where(is_z, 0, g2[:, 1])
    neigh2 = windows(st2)
    valid2 = ar[None, :] < k2[:, None]
    g_n = jnp.where(valid2, neigh2, z)
    gidx = jnp.concatenate(
        [g_n, jnp.where(is_z, z, flat)[:, None]], axis=1)
    scale0 = 1.0 / (k2.astype(jnp.float32) + 1.0)
    scale1 = 1.0 / (k1.astype(jnp.float32) + 1.0)

    qn = -(-n // NPASS)
    pieces = []
    for q in range(NPASS):
        seg = feats[q * qn:min((q + 1) * qn, n)].astype(jnp.bfloat16)
        pieces.append(jnp.zeros((8, d), jnp.bfloat16))
        pieces.append(seg)
        pieces.append(
            jnp.zeros((PASS_ROWS - 8 - seg.shape[0], d), jnp.bfloat16))
    table = jax.lax.bitcast_convert_type(
        jnp.concatenate(pieces, axis=0).reshape(-1, d // 2, 2), jnp.int32)
    gmap = jnp.where(
        gidx == z, 0,
        gidx + 8 + (gidx // qn) * (PASS_ROWS - qn))
    rw = rows // NW
    co = S
    cip = 128
    ch = rw // co
    g3 = jnp.pad(
        gmap.reshape(rows // co, co * S),
        ((0, 0), (0, cip - co * S))).reshape(NW, ch, cip)
    sums = _build_gather_sum(rows, d, ch, co, cip)(table, g3)

    w1p = jnp.take(W1, jnp.asarray(_unpack_perm(d)), axis=0)
    blk = 128
    grid = (b // blk,)
    body = functools.partial(_dense_body, S, blk)
    out = pl.pallas_call(
        body,
        grid=grid,
        in_specs=[
            pl.BlockSpec((S, blk, d), lambda i: (0, i, 0)),
            pl.BlockSpec((S, blk, 1), lambda i: (0, i, 0)),
            pl.BlockSpec((blk, 1), lambda i: (i, 0)),
            pl.BlockSpec((d, W1.shape[1]), lambda i: (0, 0)),
            pl.BlockSpec((W1.shape[1], W2.shape[1]), lambda i: (0, 0)),
        ],
        out_specs=pl.BlockSpec((blk, W2.shape[1]), lambda i: (i, 0)),
        out_shape=jax.ShapeDtypeStruct((b, W2.shape[1]), jnp.float32),
    )(sums.reshape(S, b, d), scale0.reshape(S, b, 1), scale1.reshape(b, 1),
      w1p, W2)
    return out


# speedup vs baseline: 1.3124x; 1.3124x over previous
import functools

import numpy as np

import jax
import jax.numpy as jnp
from jax import lax
from jax.experimental import pallas as pl
from jax.experimental.pallas import tpu as pltpu
from jax.experimental.pallas import tpu_sc as plsc

NN = 10
S = NN + 1
NC, NS = 2, 16
NW = NC * NS
LANES = 16

PASS_ROWS = 5120
NPASS = 2


def _build_gather_sum(rows, d, ch, co, cip):
    dw = d // 2
    mesh = plsc.VectorSubcoreMesh(
        core_axis_name="c", subcore_axis_name="s", num_cores=NC,
        num_subcores=NS)
    rw = rows // NW
    nseg = 4
    cseg = ch // nseg
    sr = rw // nseg
    stg = PASS_ROWS // NS

    @functools.partial(
        pl.kernel,
        out_type=jax.ShapeDtypeStruct((rows, d), jnp.float32),
        mesh=mesh,
        scratch_types=[
            pltpu.VMEM((ch, cip), jnp.int32),
            pltpu.VMEM((2, cip), jnp.int32),
            pltpu.VMEM((2, cip, dw), jnp.int32),
            pltpu.VMEM((sr, d), jnp.float32),
            pltpu.VMEM_SHARED((PASS_ROWS, dw), jnp.int32),
            pltpu.SemaphoreType.DMA,
        ],
        compiler_params=pltpu.CompilerParams(needs_layout_passes=False),
    )
    def gather_sum(table, g3, out, idx_v, lidx_v, rows_v, acc_v, spm, sem):
        sid = lax.axis_index("s")
        w = sid * NC + lax.axis_index("c")
        pltpu.sync_copy(g3.at[w], idx_v)

        def remap_fire(c, h):
            bsl = c & 1
            for k in range(cip // LANES):
                seg = pl.ds(k * LANES, LANES)
                v = idx_v[c, seg] - h * PASS_ROWS
                ok = (v >= 0) & (v < PASS_ROWS)
                lidx_v[bsl, seg] = jnp.where(ok, v, 0)
            pltpu.async_copy(
                spm.at[lidx_v.at[bsl]], rows_v.at[bsl], sem)

        def segment(sg, carry0):
            def zero(i, carry):
                for g in range(d // LANES):
                    acc_v[i, pl.ds(g * LANES, LANES)] = jnp.zeros(
                        (LANES,), jnp.float32)
                return carry

            lax.fori_loop(0, sr, zero, 0, unroll=False)

            def passes(h, carry):
                pltpu.sync_copy(
                    table.at[pl.ds(h * PASS_ROWS + sid * stg, stg)],
                    spm.at[pl.ds(sid * stg, stg)])
                plsc.subcore_barrier()
                remap_fire(sg * cseg, h)

                def chunk(cl, carry2):
                    c = sg * cseg + cl
                    bsl = c & 1
                    pltpu.make_async_copy(
                        table.at[pl.ds(0, cip)], rows_v.at[bsl],
                        sem).wait()

                    @pl.when(cl + 1 < cseg)
                    def _():
                        remap_fire(c + 1, h)

                    def seg_loop(g, carry3):
                        src = pl.ds(g * LANES, LANES)
                        dst_a = pl.ds(g * 2 * LANES, LANES)
                        dst_b = pl.ds(g * 2 * LANES + LANES, LANES)
                        for o in range(co):
                            b0 = o * S
                            a, bb = plsc.unpack(
                                plsc.bitcast(
                                    rows_v[bsl, b0, src], jnp.bfloat16),
                                format=plsc.PackFormat.INTERLEAVED,
                                preferred_element_type=jnp.float32)
                            for r in range(1, S):
                                ar, br = plsc.unpack(
                                    plsc.bitcast(
                                        rows_v[bsl, b0 + r, src],
                                        jnp.bfloat16),
                                    format=plsc.PackFormat.INTERLEAVED,
                                    preferred_element_type=jnp.float32)
                                a = a + ar
                                bb = bb + br
                            oi = cl * co + o
                            acc_v[oi, dst_a] = acc_v[oi, dst_a] + a
                            acc_v[oi, dst_b] = acc_v[oi, dst_b] + bb
                        return carry3

                    lax.fori_loop(0, dw // LANES, seg_loop, 0,
                                  unroll=False)
                    return carry2

                lax.fori_loop(0, cseg, chunk, 0, unroll=False)
                plsc.subcore_barrier()
                return carry

            lax.fori_loop(0, NPASS, passes, 0, unroll=False)
            pltpu.sync_copy(
                acc_v, out.at[pl.ds(w * rw + sg * sr, sr)])
            return carry0

        lax.fori_loop(0, nseg, segment, 0, unroll=False)

    return gather_sum


def _unpack_perm(d):
    perm = []
    for base in range(0, d, 32):
        perm.extend(base + 2 * j for j in range(16))
        perm.extend(base + 2 * j + 1 for j in range(16))
    return np.asarray(perm, dtype=np.int32)


def _dense_body(s, blk, sums_ref, sc0_ref, sc1_ref, w1_ref, w2_ref, out_ref):
    w1 = w1_ref[...]
    acc = jnp.zeros((blk, out_ref.shape[1]), jnp.float32)
    for j in range(s):
        m = sums_ref[j] * sc0_ref[j]
        h = jnp.maximum(jnp.dot(m, w1, preferred_element_type=jnp.float32), 0.0)
        nrm = jnp.sqrt(jnp.sum(h * h, axis=1, keepdims=True))
        acc = acc + h / jnp.maximum(nrm, 1e-12)
    mean1 = acc * sc1_ref[...]
    h2 = jnp.maximum(
        jnp.dot(mean1, w2_ref[...], preferred_element_type=jnp.float32), 0.0)
    n2 = jnp.sqrt(jnp.sum(h2 * h2, axis=1, keepdims=True))
    out_ref[...] = h2 / jnp.maximum(n2, 1e-12)


def kernel(feats, adj0, adj1, samples, W1, W2):
    n, d = feats.shape
    e = adj0.shape[0]
    b = samples.shape[0]
    rows = b * S
    z = n

    starts = jnp.concatenate(
        [jnp.zeros((1,), adj1.dtype), jnp.cumsum(adj1)[:-1]])
    ar = jnp.arange(NN, dtype=adj1.dtype)
    size1 = adj1[samples]
    k1 = jnp.minimum(size1, NN)
    idx1 = jnp.clip(starts[samples][:, None] + ar[None, :], 0, e - 1)
    neigh1 = adj0[idx1]
    valid1 = ar[None, :] < k1[:, None]
    slots = jnp.where(valid1, neigh1, z)
    node_t = jnp.concatenate([slots.T, samples[None, :]], axis=0)
    flat = node_t.reshape(-1)
    is_z = flat == z
    fc = jnp.clip(flat, 0, n - 1)
    size2 = jnp.where(is_z, 0, adj1[fc])
    k2 = jnp.minimum(size2, NN)
    st2 = jnp.where(is_z, 0, starts[fc])
    idx2 = jnp.clip(st2[:, None] + ar[None, :], 0, e - 1)
    neigh2 = adj0[idx2]
    valid2 = ar[None, :] < k2[:, None]
    g_n = jnp.where(valid2, neigh2, z)
    gidx = jnp.concatenate(
        [g_n, jnp.where(is_z, z, flat)[:, None]], axis=1)
    scale0 = 1.0 / (k2.astype(jnp.float32) + 1.0)
    scale1 = 1.0 / (k1.astype(jnp.float32) + 1.0)

    qn = -(-n // NPASS)
    pieces = []
    for q in range(NPASS):
        seg = feats[q * qn:min((q + 1) * qn, n)].astype(jnp.bfloat16)
        pieces.append(jnp.zeros((8, d), jnp.bfloat16))
        pieces.append(seg)
        pieces.append(
            jnp.zeros((PASS_ROWS - 8 - seg.shape[0], d), jnp.bfloat16))
    table = jax.lax.bitcast_convert_type(
        jnp.concatenate(pieces, axis=0).reshape(-1, d // 2, 2), jnp.int32)
    gmap = jnp.where(
        gidx == z, 0,
        gidx + 8 + (gidx // qn) * (PASS_ROWS - qn))
    rw = rows // NW
    co = S
    cip = 128
    ch = rw // co
    g3 = jnp.pad(
        gmap.reshape(rows // co, co * S),
        ((0, 0), (0, cip - co * S))).reshape(NW, ch, cip)
    sums = _build_gather_sum(rows, d, ch, co, cip)(table, g3)

    w1p = jnp.take(W1, jnp.asarray(_unpack_perm(d)), axis=0)
    blk = 128
    grid = (b // blk,)
    body = functools.partial(_dense_body, S, blk)
    out = pl.pallas_call(
        body,
        grid=grid,
        in_specs=[
            pl.BlockSpec((S, blk, d), lambda i: (0, i, 0)),
            pl.BlockSpec((S, blk, 1), lambda i: (0, i, 0)),
            pl.BlockSpec((blk, 1), lambda i: (i, 0)),
            pl.BlockSpec((d, W1.shape[1]), lambda i: (0, 0)),
            pl.BlockSpec((W1.shape[1], W2.shape[1]), lambda i: (0, 0)),
        ],
        out_specs=pl.BlockSpec((blk, W2.shape[1]), lambda i: (i, 0)),
        out_shape=jax.ShapeDtypeStruct((b, W2.shape[1]), jnp.float32),
    )(sums.reshape(S, b, d), scale0.reshape(S, b, 1), scale1.reshape(b, 1),
      w1p, W2)
    return out


# speedup vs baseline: 1.3563x; 1.0334x over previous
import functools

import numpy as np

import jax
import jax.numpy as jnp
from jax import lax
from jax.experimental import pallas as pl
from jax.experimental.pallas import tpu as pltpu
from jax.experimental.pallas import tpu_sc as plsc

NN = 10
S = NN + 1
NC, NS = 2, 16
NW = NC * NS
LANES = 16

PASS_ROWS = 5120
NPASS = 2


def _build_gather_sum(rows, d, ch, co, cip):
    dw = d // 2
    mesh = plsc.VectorSubcoreMesh(
        core_axis_name="c", subcore_axis_name="s", num_cores=NC,
        num_subcores=NS)
    rw = rows // NW
    nseg = 2
    cseg = ch // nseg
    sr = rw // nseg
    stg = PASS_ROWS // NS

    @functools.partial(
        pl.kernel,
        out_type=jax.ShapeDtypeStruct((rows, d), jnp.float32),
        mesh=mesh,
        scratch_types=[
            pltpu.VMEM((ch, cip), jnp.int32),
            pltpu.VMEM((2, cip), jnp.int32),
            pltpu.VMEM((2, cip, dw), jnp.int32),
            pltpu.VMEM((sr, d), jnp.float32),
            pltpu.VMEM_SHARED((PASS_ROWS, dw), jnp.int32),
            pltpu.SemaphoreType.DMA,
        ],
        compiler_params=pltpu.CompilerParams(needs_layout_passes=False),
    )
    def gather_sum(table, g3, out, idx_v, lidx_v, rows_v, acc_v, spm, sem):
        sid = lax.axis_index("s")
        w = sid * NC + lax.axis_index("c")
        pltpu.sync_copy(g3.at[w], idx_v)

        def remap_fire(c, h):
            bsl = c & 1
            for k in range(cip // LANES):
                seg = pl.ds(k * LANES, LANES)
                v = idx_v[c, seg] - h * PASS_ROWS
                ok = (v >= 0) & (v < PASS_ROWS)
                lidx_v[bsl, seg] = jnp.where(ok, v, 0)
            pltpu.async_copy(
                spm.at[lidx_v.at[bsl]], rows_v.at[bsl], sem)

        def segment(sg, carry0):
            def zero(i, carry):
                for g in range(d // LANES):
                    acc_v[i, pl.ds(g * LANES, LANES)] = jnp.zeros(
                        (LANES,), jnp.float32)
                return carry

            lax.fori_loop(0, sr, zero, 0, unroll=False)

            def passes(h, carry):
                pltpu.sync_copy(
                    table.at[pl.ds(h * PASS_ROWS + sid * stg, stg)],
                    spm.at[pl.ds(sid * stg, stg)])
                plsc.subcore_barrier()
                remap_fire(sg * cseg, h)

                def chunk(cl, carry2):
                    c = sg * cseg + cl
                    bsl = c & 1
                    pltpu.make_async_copy(
                        table.at[pl.ds(0, cip)], rows_v.at[bsl],
                        sem).wait()

                    @pl.when(cl + 1 < cseg)
                    def _():
                        remap_fire(c + 1, h)

                    def seg_loop(g, carry3):
                        src = pl.ds(g * LANES, LANES)
                        dst_a = pl.ds(g * 2 * LANES, LANES)
                        dst_b = pl.ds(g * 2 * LANES + LANES, LANES)
                        for o in range(co):
                            b0 = o * S
                            a, bb = plsc.unpack(
                                plsc.bitcast(
                                    rows_v[bsl, b0, src], jnp.bfloat16),
                                format=plsc.PackFormat.INTERLEAVED,
                                preferred_element_type=jnp.float32)
                            for r in range(1, S):
                                ar, br = plsc.unpack(
                                    plsc.bitcast(
                                        rows_v[bsl, b0 + r, src],
                                        jnp.bfloat16),
                                    format=plsc.PackFormat.INTERLEAVED,
                                    preferred_element_type=jnp.float32)
                                a = a + ar
                                bb = bb + br
                            oi = cl * co + o
                            acc_v[oi, dst_a] = acc_v[oi, dst_a] + a
                            acc_v[oi, dst_b] = acc_v[oi, dst_b] + bb
                        return carry3

                    lax.fori_loop(0, dw // LANES, seg_loop, 0,
                                  unroll=False)
                    return carry2

                lax.fori_loop(0, cseg, chunk, 0, unroll=False)
                plsc.subcore_barrier()
                return carry

            lax.fori_loop(0, NPASS, passes, 0, unroll=False)
            pltpu.sync_copy(
                acc_v, out.at[pl.ds(w * rw + sg * sr, sr)])
            return carry0

        lax.fori_loop(0, nseg, segment, 0, unroll=False)

    return gather_sum


def _unpack_perm(d):
    perm = []
    for base in range(0, d, 32):
        perm.extend(base + 2 * j for j in range(16))
        perm.extend(base + 2 * j + 1 for j in range(16))
    return np.asarray(perm, dtype=np.int32)


def _dense_body(s, blk, sums_ref, sc0_ref, sc1_ref, w1_ref, w2_ref, out_ref):
    w1 = w1_ref[...]
    acc = jnp.zeros((blk, out_ref.shape[1]), jnp.float32)
    for j in range(s):
        m = sums_ref[j] * sc0_ref[j]
        h = jnp.maximum(jnp.dot(m, w1, preferred_element_type=jnp.float32), 0.0)
        nrm = jnp.sqrt(jnp.sum(h * h, axis=1, keepdims=True))
        acc = acc + h / jnp.maximum(nrm, 1e-12)
    mean1 = acc * sc1_ref[...]
    h2 = jnp.maximum(
        jnp.dot(mean1, w2_ref[...], preferred_element_type=jnp.float32), 0.0)
    n2 = jnp.sqrt(jnp.sum(h2 * h2, axis=1, keepdims=True))
    out_ref[...] = h2 / jnp.maximum(n2, 1e-12)


def kernel(feats, adj0, adj1, samples, W1, W2):
    n, d = feats.shape
    e = adj0.shape[0]
    b = samples.shape[0]
    rows = b * S
    z = n

    starts = jnp.concatenate(
        [jnp.zeros((1,), adj1.dtype), jnp.cumsum(adj1)[:-1]])
    ar = jnp.arange(NN, dtype=adj1.dtype)
    size1 = adj1[samples]
    k1 = jnp.minimum(size1, NN)
    idx1 = jnp.clip(starts[samples][:, None] + ar[None, :], 0, e - 1)
    neigh1 = adj0[idx1]
    valid1 = ar[None, :] < k1[:, None]
    slots = jnp.where(valid1, neigh1, z)
    node_t = jnp.concatenate([slots.T, samples[None, :]], axis=0)
    flat = node_t.reshape(-1)
    is_z = flat == z
    fc = jnp.clip(flat, 0, n - 1)
    size2 = jnp.where(is_z, 0, adj1[fc])
    k2 = jnp.minimum(size2, NN)
    st2 = jnp.where(is_z, 0, starts[fc])
    idx2 = jnp.clip(st2[:, None] + ar[None, :], 0, e - 1)
    neigh2 = adj0[idx2]
    valid2 = ar[None, :] < k2[:, None]
    g_n = jnp.where(valid2, neigh2, z)
    gidx = jnp.concatenate(
        [g_n, jnp.where(is_z, z, flat)[:, None]], axis=1)
    scale0 = 1.0 / (k2.astype(jnp.float32) + 1.0)
    scale1 = 1.0 / (k1.astype(jnp.float32) + 1.0)

    qn = -(-n // NPASS)
    pieces = []
    for q in range(NPASS):
        seg = feats[q * qn:min((q + 1) * qn, n)].astype(jnp.bfloat16)
        pieces.append(jnp.zeros((8, d), jnp.bfloat16))
        pieces.append(seg)
        pieces.append(
            jnp.zeros((PASS_ROWS - 8 - seg.shape[0], d), jnp.bfloat16))
    table = jax.lax.bitcast_convert_type(
        jnp.concatenate(pieces, axis=0).reshape(-1, d // 2, 2), jnp.int32)
    gmap = jnp.where(
        gidx == z, 0,
        gidx + 8 + (gidx // qn) * (PASS_ROWS - qn))
    rw = rows // NW
    co = S
    cip = 128
    ch = rw // co
    g3 = jnp.pad(
        gmap.reshape(rows // co, co * S),
        ((0, 0), (0, cip - co * S))).reshape(NW, ch, cip)
    sums = _build_gather_sum(rows, d, ch, co, cip)(table, g3)

    w1p = jnp.take(W1, jnp.asarray(_unpack_perm(d)), axis=0)
    blk = 128
    grid = (b // blk,)
    body = functools.partial(_dense_body, S, blk)
    out = pl.pallas_call(
        body,
        grid=grid,
        in_specs=[
            pl.BlockSpec((S, blk, d), lambda i: (0, i, 0)),
            pl.BlockSpec((S, blk, 1), lambda i: (0, i, 0)),
            pl.BlockSpec((blk, 1), lambda i: (i, 0)),
            pl.BlockSpec((d, W1.shape[1]), lambda i: (0, 0)),
            pl.BlockSpec((W1.shape[1], W2.shape[1]), lambda i: (0, 0)),
        ],
        out_specs=pl.BlockSpec((blk, W2.shape[1]), lambda i: (i, 0)),
        out_shape=jax.ShapeDtypeStruct((b, W2.shape[1]), jnp.float32),
    )(sums.reshape(S, b, d), scale0.reshape(S, b, 1), scale1.reshape(b, 1),
      w1p, W2)
    return out
